# baseline (speedup 1.0000x reference)
import jax
import jax.numpy as jnp
from jax.experimental import pallas as pl

_RANGE_MIN = 24
_RANGE_MAX = 31
_IN_DIM = 128
_EMB_DIM = 18


def _body(x_ref, o_ref):
    v = x_ref[...]
    ci = jax.lax.broadcasted_iota(jnp.int32, v.shape, 1)
    is_card = (ci >= _RANGE_MIN) & (ci < _RANGE_MAX)
    vi = jnp.floor(v)
    r = jnp.floor(vi * 0.25)
    s = vi - 4.0 * r
    one = jnp.ones_like(v)
    zero = jnp.zeros_like(v)
    for e in range(_EMB_DIM):
        if e < 13:
            bit = jnp.where(r == float(e), one, zero)
        elif e < 17:
            bit = jnp.where(s == float(e - 13), one, zero)
        else:
            bit = one
        o_ref[e] = jnp.where(is_card, bit, v)


@jax.jit
def _run(x2):
    b = x2.shape[0]
    bb = 512
    out = pl.pallas_call(
        _body,
        grid=(b // bb,),
        in_specs=[pl.BlockSpec((bb, _IN_DIM), lambda i: (i, 0))],
        out_specs=pl.BlockSpec((_EMB_DIM, bb, _IN_DIM), lambda i: (0, i, 0)),
        out_shape=jax.ShapeDtypeStruct((_EMB_DIM, b, _IN_DIM), jnp.float32),
    )(x2)
    return out.transpose(1, 2, 0)


def kernel(x):
    if x.ndim == 3:
        x = x[:, 0, :]
    return _run(x)


# speedup vs baseline: 1.0683x; 1.0683x over previous
import jax
import jax.numpy as jnp
from jax.experimental import pallas as pl

_RANGE_MIN = 24
_RANGE_MAX = 31
_IN_DIM = 128
_EMB_DIM = 18


def _body(x_ref, o_ref):
    v = x_ref[...]
    ci = jax.lax.broadcasted_iota(jnp.int32, v.shape, 1)
    is_card = (ci >= _RANGE_MIN) & (ci < _RANGE_MAX)
    vi = jnp.floor(v)
    r = jnp.floor(vi * 0.25)
    s = vi - 4.0 * r
    one = jnp.ones_like(v)
    zero = jnp.zeros_like(v)
    for e in range(_EMB_DIM):
        if e < 13:
            bit = jnp.where(r == float(e), one, zero)
        elif e < 17:
            bit = jnp.where(s == float(e - 13), one, zero)
        else:
            bit = one
        o_ref[e] = jnp.where(is_card, bit, v)


@jax.jit
def _run(x2):
    b = x2.shape[0]
    bb = 1024
    out = pl.pallas_call(
        _body,
        grid=(b // bb,),
        in_specs=[pl.BlockSpec((bb, _IN_DIM), lambda i: (i, 0))],
        out_specs=pl.BlockSpec((_EMB_DIM, bb, _IN_DIM), lambda i: (0, i, 0)),
        out_shape=jax.ShapeDtypeStruct((_EMB_DIM, b, _IN_DIM), jnp.float32),
    )(x2)
    return out.transpose(1, 2, 0)


def kernel(x):
    if x.ndim == 3:
        x = x[:, 0, :]
    return _run(x)
